# baseline (speedup 1.0000x reference)
import functools

import jax
import jax.numpy as jnp
from jax import lax
from jax.experimental import pallas as pl
from jax.experimental.pallas import tpu as pltpu
from jax.experimental.pallas import tpu_sc as plsc

B = 16384
NC, NS = 2, 16
NW = NC * NS
BPW = B // NW
NGRP = BPW // 16
NCHUNK = BPW // 128
ISTREAM = 4
IGRP0 = ISTREAM * 8
D = 16
NU, NI, NG = 1000, 500, 48
TAIL = NI + NG

_mesh = plsc.VectorSubcoreMesh(core_axis_name="c", subcore_axis_name="s")


@functools.partial(
    pl.kernel,
    mesh=_mesh,
    compiler_params=pltpu.CompilerParams(
        use_tc_tiling_on_sc=False, needs_layout_passes=False),
    out_type=[
        jax.ShapeDtypeStruct((B, D), jnp.float32),
        jax.ShapeDtypeStruct((B, D), jnp.float32),
        jax.ShapeDtypeStruct((B, D), jnp.float32),
    ],
    scratch_types=[
        pltpu.VMEM((NCHUNK, 128), jnp.int32),
        pltpu.VMEM((ISTREAM, 128), jnp.int32),
        pltpu.VMEM((NCHUNK, 128), jnp.int32),
        pltpu.VMEM((BPW,), jnp.int32),
        pltpu.VMEM((BPW,), jnp.int32),
        pltpu.VMEM((BPW,), jnp.int32),
        pltpu.VMEM((BPW, D), jnp.float32),
        pltpu.VMEM((BPW, D), jnp.float32),
        pltpu.VMEM((BPW, D), jnp.float32),
        pltpu.SemaphoreType.DMA,
        pltpu.SemaphoreType.DMA,
        pltpu.SemaphoreType.DMA,
    ],
)
def _sc_gather(uid_hbm, iid_hbm, g_hbm, h_hbm, table,
               out_u, out_i, out_gh,
               idx_u, idx_is, idx_gs, idx_i, g_v, h_v,
               rows_u, rows_i, rows_gh, sem_idx, sem_gu, sem_store):
    wid = lax.axis_index("s") * NC + lax.axis_index("c")
    base = wid * BPW
    s = pl.ds(base, BPW)
    icps = [pltpu.async_copy(iid_hbm.at[s], idx_i, sem_idx),
            pltpu.async_copy(g_hbm.at[s], g_v, sem_idx),
            pltpu.async_copy(h_hbm.at[s], h_v, sem_idx)]
    for j in range(NCHUNK):
        icps.append(pltpu.async_copy(
            uid_hbm.at[pl.ds(base + j * 128, 128)], idx_u.at[j], sem_idx))
    for cp in icps:
        cp.wait()
    gcps = []
    for j in range(NCHUNK):
        gcps.append(pltpu.async_copy(
            table.at[idx_u.at[j]], rows_u.at[pl.ds(j * 128, 128)], sem_gu))
    for j in range(ISTREAM):
        for t in range(8):
            sl = pl.ds(j * 128 + t * 16, 16)
            idx_is[j, pl.ds(t * 16, 16)] = idx_i[sl] + NU
    for j in range(NCHUNK):
        for t in range(8):
            sl = pl.ds(j * 128 + t * 16, 16)
            idx_gs[j, pl.ds(t * 16, 16)] = NU + NI + g_v[sl] * 24 + h_v[sl]
    for j in range(ISTREAM):
        gcps.append(pltpu.async_copy(
            table.at[idx_is.at[j]], rows_i.at[pl.ds(j * 128, 128)], sem_gu))
    for j in range(NCHUNK):
        gcps.append(pltpu.async_copy(
            table.at[idx_gs.at[j]], rows_gh.at[pl.ds(j * 128, 128)], sem_gu))
    for cp in gcps:
        cp.wait()
    o = pl.ds(base, BPW)
    scps = [pltpu.async_copy(rows_u, out_u.at[o], sem_store),
            pltpu.async_copy(rows_i, out_i.at[o], sem_store),
            pltpu.async_copy(rows_gh, out_gh.at[o], sem_store)]
    for cp in scps:
        cp.wait()


ROWS = B // 8
RBLK = 1024


def _mlp_body(u_ref, i_ref, gh_ref, w1u_ref, w1i_ref, w1gh_ref, b1_ref,
              w2_ref, b2_ref, out_ref):
    b1v = b1_ref[...]
    b1big = jnp.concatenate([b1v] * 8, axis=1)
    y = (jnp.dot(u_ref[...], w1u_ref[...], preferred_element_type=jnp.float32)
         + jnp.dot(i_ref[...], w1i_ref[...], preferred_element_type=jnp.float32)
         + jnp.dot(gh_ref[...], w1gh_ref[...], preferred_element_type=jnp.float32)
         + b1big)
    y = jnp.maximum(y, 0.0)
    o = jnp.dot(y, w2_ref[...], preferred_element_type=jnp.float32) + b2_ref[...]
    out_ref[...] = o


_mlp = pl.pallas_call(
    _mlp_body,
    grid=(ROWS // RBLK,),
    in_specs=[
        pl.BlockSpec((RBLK, 128), lambda k: (k, 0)),
        pl.BlockSpec((RBLK, 128), lambda k: (k, 0)),
        pl.BlockSpec((RBLK, 128), lambda k: (k, 0)),
        pl.BlockSpec((128, 256), lambda k: (0, 0)),
        pl.BlockSpec((128, 256), lambda k: (0, 0)),
        pl.BlockSpec((128, 256), lambda k: (0, 0)),
        pl.BlockSpec((1, 32), lambda k: (0, 0)),
        pl.BlockSpec((256, 8), lambda k: (0, 0)),
        pl.BlockSpec((1, 1), lambda k: (0, 0)),
    ],
    out_specs=pl.BlockSpec((RBLK, 8), lambda k: (k, 0)),
    out_shape=jax.ShapeDtypeStruct((ROWS, 8), jnp.float32),
)


def kernel(user_id, item_id, gender, hour, user_emb, item_emb, gender_emb,
           hour_emb, W1, b1, W2, b2):
    uid = user_id.astype(jnp.int32)
    iid = item_id.astype(jnp.int32)
    g = gender.astype(jnp.int32)
    h = hour.astype(jnp.int32)
    ar = jnp.arange(48)
    gh_table = jnp.concatenate(
        [jnp.take(gender_emb, ar // 24, axis=0),
         jnp.take(hour_emb, ar % 24, axis=0),
         jnp.zeros((48, 8), jnp.float32)], axis=1)
    table = jnp.concatenate([user_emb, item_emb, gh_table], axis=0)
    u, i, gh = _sc_gather(uid, iid, g, h, table)
    eye8 = jnp.eye(8, dtype=jnp.float32)
    w1gh = jnp.concatenate([W1[32:40], jnp.zeros((8, 32), W1.dtype)], axis=0)
    w1u_big = jnp.kron(eye8, W1[0:16])
    w1i_big = jnp.kron(eye8, W1[16:32])
    w1gh_big = jnp.kron(eye8, w1gh)
    w2_big = jnp.kron(eye8, W2)
    out = _mlp(u.reshape(ROWS, 128), i.reshape(ROWS, 128), gh.reshape(ROWS, 128),
               w1u_big, w1i_big, w1gh_big, b1.reshape(1, 32), w2_big,
               b2.reshape(1, 1))
    return out.reshape(B, 1)


# speedup vs baseline: 1.3386x; 1.3386x over previous
import functools

import jax
import jax.numpy as jnp
from jax import lax
from jax.experimental import pallas as pl
from jax.experimental.pallas import tpu as pltpu
from jax.experimental.pallas import tpu_sc as plsc

B = 16384
NC, NS = 2, 16
NW = NC * NS
BPW = B // NW
NGRP = BPW // 16
NCHUNK = BPW // 128
ISTREAM = 4
IGRP0 = ISTREAM * 8
D = 16
NU, NI, NG = 1000, 500, 48
TAIL = NI + NG

_mesh = plsc.VectorSubcoreMesh(core_axis_name="c", subcore_axis_name="s")


@functools.partial(
    pl.kernel,
    mesh=_mesh,
    compiler_params=pltpu.CompilerParams(
        use_tc_tiling_on_sc=False, needs_layout_passes=False),
    out_type=[
        jax.ShapeDtypeStruct((B, D), jnp.float32),
        jax.ShapeDtypeStruct((B, D), jnp.float32),
        jax.ShapeDtypeStruct((B, D), jnp.float32),
    ],
    scratch_types=[
        pltpu.VMEM((NCHUNK, 128), jnp.int32),
        pltpu.VMEM((ISTREAM, 128), jnp.int32),
        pltpu.VMEM((BPW,), jnp.int32),
        pltpu.VMEM((BPW,), jnp.int32),
        pltpu.VMEM((BPW,), jnp.int32),
        pltpu.VMEM((NG, D), jnp.float32),
        pltpu.VMEM((BPW, D), jnp.float32),
        pltpu.VMEM((BPW, D), jnp.float32),
        pltpu.VMEM((BPW, D), jnp.float32),
        pltpu.SemaphoreType.DMA,
        pltpu.SemaphoreType.DMA,
        pltpu.SemaphoreType.DMA,
        pltpu.SemaphoreType.DMA,
    ],
)
def _sc_gather(uid_hbm, iid_hbm, g_hbm, h_hbm, table,
               out_u, out_i, out_gh,
               idx_u, idx_is, idx_i, g_v, h_v, tab_g,
               rows_u, rows_i, rows_gh, sem_idx, sem_tab, sem_gu, sem_store):
    wid = lax.axis_index("s") * NC + lax.axis_index("c")
    base = wid * BPW
    s = pl.ds(base, BPW)
    icps = [pltpu.async_copy(iid_hbm.at[s], idx_i, sem_idx),
            pltpu.async_copy(g_hbm.at[s], g_v, sem_idx),
            pltpu.async_copy(h_hbm.at[s], h_v, sem_idx)]
    for j in range(NCHUNK):
        icps.append(pltpu.async_copy(
            uid_hbm.at[pl.ds(base + j * 128, 128)], idx_u.at[j], sem_idx))
    tcp = pltpu.async_copy(table.at[pl.ds(NU + NI, NG)], tab_g, sem_tab)
    for cp in icps:
        cp.wait()
    gcps = []
    for j in range(NCHUNK):
        gcps.append(pltpu.async_copy(
            table.at[idx_u.at[j]], rows_u.at[pl.ds(j * 128, 128)], sem_gu))
    for j in range(ISTREAM):
        for t in range(8):
            sl = pl.ds(j * 128 + t * 16, 16)
            idx_is[j, pl.ds(t * 16, 16)] = idx_i[sl] + NU
    for j in range(ISTREAM):
        gcps.append(pltpu.async_copy(
            table.at[idx_is.at[j]], rows_i.at[pl.ds(j * 128, 128)], sem_gu))
    tcp.wait()
    iota16 = lax.iota(jnp.int32, 16)
    cols = [jnp.full((16,), c, jnp.int32) for c in range(D)]

    @plsc.parallel_loop(0, NGRP, unroll=4)
    def _loop_g(gi):
        off = gi * 16
        sl = pl.ds(off, 16)
        smp = off + iota16
        rg = g_v[sl] * 24 + h_v[sl]
        for c in range(D):
            plsc.store_scatter(rows_gh, [smp, cols[c]],
                               plsc.load_gather(tab_g, [rg, cols[c]]))

    for cp in gcps:
        cp.wait()
    o = pl.ds(base, BPW)
    scps = [pltpu.async_copy(rows_u, out_u.at[o], sem_store),
            pltpu.async_copy(rows_i, out_i.at[o], sem_store),
            pltpu.async_copy(rows_gh, out_gh.at[o], sem_store)]
    for cp in scps:
        cp.wait()


ROWS = B // 8
RBLK = 1024


def _mlp_body(u_ref, i_ref, gh_ref, w1u_ref, w1i_ref, w1gh_ref, b1_ref,
              w2_ref, b2_ref, out_ref):
    b1v = b1_ref[...]
    b1big = jnp.concatenate([b1v] * 8, axis=1)
    y = (jnp.dot(u_ref[...], w1u_ref[...], preferred_element_type=jnp.float32)
         + jnp.dot(i_ref[...], w1i_ref[...], preferred_element_type=jnp.float32)
         + jnp.dot(gh_ref[...], w1gh_ref[...], preferred_element_type=jnp.float32)
         + b1big)
    y = jnp.maximum(y, 0.0)
    o = jnp.dot(y, w2_ref[...], preferred_element_type=jnp.float32) + b2_ref[...]
    out_ref[...] = o


_mlp = pl.pallas_call(
    _mlp_body,
    grid=(ROWS // RBLK,),
    in_specs=[
        pl.BlockSpec((RBLK, 128), lambda k: (k, 0)),
        pl.BlockSpec((RBLK, 128), lambda k: (k, 0)),
        pl.BlockSpec((RBLK, 128), lambda k: (k, 0)),
        pl.BlockSpec((128, 256), lambda k: (0, 0)),
        pl.BlockSpec((128, 256), lambda k: (0, 0)),
        pl.BlockSpec((128, 256), lambda k: (0, 0)),
        pl.BlockSpec((1, 32), lambda k: (0, 0)),
        pl.BlockSpec((256, 8), lambda k: (0, 0)),
        pl.BlockSpec((1, 1), lambda k: (0, 0)),
    ],
    out_specs=pl.BlockSpec((RBLK, 8), lambda k: (k, 0)),
    out_shape=jax.ShapeDtypeStruct((ROWS, 8), jnp.float32),
)


def kernel(user_id, item_id, gender, hour, user_emb, item_emb, gender_emb,
           hour_emb, W1, b1, W2, b2):
    uid = user_id.astype(jnp.int32)
    iid = item_id.astype(jnp.int32)
    g = gender.astype(jnp.int32)
    h = hour.astype(jnp.int32)
    ar = jnp.arange(48)
    gh_table = jnp.concatenate(
        [jnp.take(gender_emb, ar // 24, axis=0),
         jnp.take(hour_emb, ar % 24, axis=0),
         jnp.zeros((48, 8), jnp.float32)], axis=1)
    table = jnp.concatenate([user_emb, item_emb, gh_table], axis=0)
    u, i, gh = _sc_gather(uid, iid, g, h, table)
    eye8 = jnp.eye(8, dtype=jnp.float32)
    w1gh = jnp.concatenate([W1[32:40], jnp.zeros((8, 32), W1.dtype)], axis=0)
    w1u_big = jnp.kron(eye8, W1[0:16])
    w1i_big = jnp.kron(eye8, W1[16:32])
    w1gh_big = jnp.kron(eye8, w1gh)
    w2_big = jnp.kron(eye8, W2)
    out = _mlp(u.reshape(ROWS, 128), i.reshape(ROWS, 128), gh.reshape(ROWS, 128),
               w1u_big, w1i_big, w1gh_big, b1.reshape(1, 32), w2_big,
               b2.reshape(1, 1))
    return out.reshape(B, 1)
